# baseline (speedup 1.0000x reference)
import functools

import jax
import jax.numpy as jnp
from jax import lax
from jax.experimental import pallas as pl
from jax.experimental.pallas import tpu as pltpu
from jax.experimental.pallas import tpu_sc as plsc


def _chunks_of_L(L):
    chunks = []
    off = 0
    while off < L:
        size = min(128, L - off)
        if L - off > 128:
            size -= size % 8
        chunks.append((off, size))
        off += size
    return chunks


def _make_sc_sum(B, L, V, D, NW):
    assert B % NW == 0
    dpw = B // NW
    assert (dpw * L) % 8 == 0 and (L % 8) == 0
    chunks = _chunks_of_L(L)
    mesh = plsc.VectorSubcoreMesh(core_axis_name="c", subcore_axis_name="s")
    NC = mesh.num_cores

    @functools.partial(
        pl.kernel,
        out_type=jax.ShapeDtypeStruct((B, D), jnp.float32),
        mesh=mesh,
        compiler_params=pltpu.CompilerParams(use_tc_tiling_on_sc=False),
        scratch_types=[
            pltpu.VMEM((dpw * L,), jnp.int32),
            pltpu.VMEM((L, D), jnp.float32),
            pltpu.VMEM((L, D), jnp.float32),
            pltpu.VMEM((dpw, D), jnp.float32),
            pltpu.SemaphoreType.DMA,
            pltpu.SemaphoreType.DMA,
        ],
    )
    def sc_sum(tokens_hbm, table_hbm, out_hbm, idx_v, buf_a, buf_b, outblk, sem_a, sem_b):
        wid = lax.axis_index("s") * NC + lax.axis_index("c")
        base_doc = wid * dpw

        pltpu.sync_copy(tokens_hbm.at[pl.ds(base_doc * L, dpw * L)], idx_v)

        def gather_start(d, buf, sem):
            off = d * L
            for c_off, c_sz in chunks:
                pltpu.async_copy(
                    table_hbm.at[idx_v.at[pl.ds(off + c_off, c_sz)]],
                    buf.at[pl.ds(c_off, c_sz)],
                    sem,
                )

        def gather_wait(buf, sem):
            for c_off, c_sz in chunks:
                pltpu.make_async_copy(
                    table_hbm.at[idx_v.at[pl.ds(c_off, c_sz)]],
                    buf.at[pl.ds(c_off, c_sz)],
                    sem,
                ).wait()

        n_groups = D // 16
        UNROLL = 8
        assert L % UNROLL == 0

        def reduce_doc(buf, d):
            zero = jnp.zeros((16,), jnp.float32)

            def body(t0, accs):
                accs = list(accs)
                for j in range(UNROLL):
                    t = t0 * UNROLL + j
                    for g in range(n_groups):
                        accs[g] = accs[g] + buf[t, pl.ds(g * 16, 16)]
                return tuple(accs)

            accs = lax.fori_loop(0, L // UNROLL, body, (zero,) * n_groups)
            for g in range(n_groups):
                outblk[d, pl.ds(g * 16, 16)] = accs[g]

        gather_start(0, buf_a, sem_a)

        def pair_body(i, _):
            d0 = 2 * i
            gather_start(d0 + 1, buf_b, sem_b)
            gather_wait(buf_a, sem_a)
            reduce_doc(buf_a, d0)

            @pl.when(d0 + 2 < dpw)
            def _():
                gather_start(d0 + 2, buf_a, sem_a)

            gather_wait(buf_b, sem_b)
            reduce_doc(buf_b, d0 + 1)
            return 0

        lax.fori_loop(0, dpw // 2, pair_body, 0)

        pltpu.sync_copy(outblk, out_hbm.at[pl.ds(base_doc, dpw)])

    return sc_sum


def _linear_body(sums_ref, len_ref, w_ref, b_ref, out_ref):
    inv = 1.0 / jnp.maximum(len_ref[...], 1).astype(jnp.float32)
    emb = sums_ref[...] * inv
    out_ref[...] = (
        jnp.dot(emb, w_ref[...], preferred_element_type=jnp.float32) + b_ref[...]
    )


def _tc_linear(sums, lens2d, W, b2d, BLK=512):
    B, D = sums.shape
    NL = W.shape[1]
    return pl.pallas_call(
        _linear_body,
        grid=(B // BLK,),
        in_specs=[
            pl.BlockSpec((BLK, D), lambda i: (i, 0)),
            pl.BlockSpec((BLK, 1), lambda i: (i, 0)),
            pl.BlockSpec((D, NL), lambda i: (0, 0)),
            pl.BlockSpec((1, NL), lambda i: (0, 0)),
        ],
        out_specs=pl.BlockSpec((BLK, NL), lambda i: (i, 0)),
        out_shape=jax.ShapeDtypeStruct((B, NL), jnp.float32),
    )(sums, lens2d, W, b2d)


@jax.jit
def kernel(doc_token, doc_token_len, token_table, W, b):
    B, L = doc_token.shape
    V, D = token_table.shape
    NW = 32
    sc_sum = _make_sc_sum(B, L, V, D, NW)
    tokens = doc_token.reshape(-1).astype(jnp.int32)
    sums = sc_sum(tokens, token_table)
    lens2d = doc_token_len.reshape(B, 1)
    b2d = b.reshape(1, -1)
    return _tc_linear(sums, lens2d, W, b2d)
